# Initial kernel scaffold; baseline (speedup 1.0000x reference)
import jax
import jax.numpy as jnp
from jax.experimental import pallas as pl


def kernel(outputs, output_symbols, targets, mask):
    raise NotImplementedError("write your pallas kernel here")


# speedup vs baseline: 5.9821x; 5.9821x over previous
import functools
import math

import jax
import jax.numpy as jnp
from jax import lax
from jax.experimental import pallas as pl
from jax.experimental.pallas import tpu as pltpu

_TEMP = 2.0
_E = math.exp(1.0 / _TEMP)
_BIG = 1e9


def _dp_hits(os_, tg, mf, B, P, L):
    jidx = lax.broadcasted_iota(jnp.int32, (B, L), 1).astype(jnp.float32)
    row = jidx
    hits = []
    for i in range(P):
        if i > 0:
            sym = os_[:, i - 1 : i]
            neq = (sym != tg).astype(jnp.float32)
            e_rest = jnp.minimum(row[:, :-1] + neq[:, :-1], row[:, 1:] + 1.0)
            e0 = jnp.full((B, 1), float(i), dtype=jnp.float32)
            e = jnp.concatenate([e0, e_rest], axis=1)
            m = e - jidx
            s = 1
            while s < L:
                shifted = jnp.concatenate(
                    [jnp.full((B, s), _BIG, dtype=jnp.float32), m[:, :-s]], axis=1
                )
                m = jnp.minimum(m, shifted)
                s *= 2
            row = m + jidx
        dm = jnp.where(mf > 0.0, row, _BIG)
        mn = jnp.min(dm, axis=1, keepdims=True)
        hit = ((dm == mn) & (mf > 0.0)).astype(jnp.float32)
        hits.append(hit)
    return jnp.stack(hits, axis=1)


def _body(osym_ref, tgt_ref, tcol_ref, maskf_ref, out_block_ref,
          o_ref, hit_ref, acc_ref, *, B, P, L, V):
    b = pl.program_id(0)

    @pl.when(b == 0)
    def _init():
        os_ = osym_ref[...]
        tg = tgt_ref[...]
        mf = maskf_ref[...]
        hit_ref[...] = _dp_hits(os_, tg, mf, B, P, L)
        acc_ref[0] = 0.0
        acc_ref[1] = 0.0

    x = out_block_ref[0]
    sall = jnp.sum(x, axis=1, keepdims=True)

    tcol = tcol_ref[0]
    vio = lax.broadcasted_iota(jnp.int32, (L, V), 1)
    oneh = (vio == tcol).astype(jnp.float32)
    hit_b = hit_ref[b]
    ind = jnp.minimum(
        jax.lax.dot(hit_b, oneh, preferred_element_type=jnp.float32), 1.0
    )
    shit = jnp.sum(x * ind, axis=1, keepdims=True)
    kk = jnp.sum(ind, axis=1, keepdims=True)

    z = kk * _E + (float(V) - kk)
    kl = 0.5 * _E * kk / z - jnp.log(z) - ((_E - 1.0) * shit + sall) / z

    mrow = maskf_ref[pl.ds(b, 1), :]
    w = jnp.sum(mrow)
    per_b = jnp.sum(kl[:, 0] * mrow[0]) / (w + 1e-13)
    acc_ref[0] += per_b
    acc_ref[1] += (w > 0.0).astype(jnp.float32)

    @pl.when(b == pl.num_programs(0) - 1)
    def _fin():
        val = acc_ref[0] / (acc_ref[1] + 1e-13)
        o_ref[...] = jnp.full((1, 1), 0.0, jnp.float32) + val


def kernel(outputs, output_symbols, targets, mask):
    B, P, V = outputs.shape
    L = targets.shape[1]
    maskf = mask.astype(jnp.float32)
    targets_col = targets[:, :, None]

    body = functools.partial(_body, B=B, P=P, L=L, V=V)
    out = pl.pallas_call(
        body,
        grid=(B,),
        in_specs=[
            pl.BlockSpec((B, P), lambda b: (0, 0)),
            pl.BlockSpec((B, L), lambda b: (0, 0)),
            pl.BlockSpec((1, L, 1), lambda b: (b, 0, 0)),
            pl.BlockSpec((B, L), lambda b: (0, 0)),
            pl.BlockSpec((1, P, V), lambda b: (b, 0, 0)),
        ],
        out_specs=pl.BlockSpec((1, 1), lambda b: (0, 0)),
        out_shape=jax.ShapeDtypeStruct((1, 1), jnp.float32),
        scratch_shapes=[
            pltpu.VMEM((B, P, L), jnp.float32),
            pltpu.SMEM((2,), jnp.float32),
        ],
    )(output_symbols, targets, targets_col, maskf, outputs)
    return out[0, 0]
